# Initial kernel scaffold; baseline (speedup 1.0000x reference)
import jax
import jax.numpy as jnp
from jax.experimental import pallas as pl


def kernel(x, cb0, cb1, cb2, cb3):
    raise NotImplementedError("write your pallas kernel here")


# speedup vs baseline: 18.7358x; 18.7358x over previous
import functools

import jax
import jax.numpy as jnp
from jax.experimental import pallas as pl

N = 1024
V = 1024
D = 256
NUM_STAGES = 4
BN = 256


def _rvq_kernel(x_ref, cb0_ref, cb1_ref, cb2_ref, cb3_ref,
                codes_ref, quant_ref, resid_ref):
    r = x_ref[...]
    quant = jnp.zeros_like(r)
    cb_refs = (cb0_ref, cb1_ref, cb2_ref, cb3_ref)
    for k in range(NUM_STAGES):
        cb = cb_refs[k][...]
        cbn = jnp.sum(cb * cb, axis=1, keepdims=True)
        dots = jax.lax.dot_general(
            r, cb, (((1,), (1,)), ((), ())),
            precision=jax.lax.Precision.HIGHEST,
            preferred_element_type=jnp.float32)
        scores = jnp.transpose(cbn) - 2.0 * dots
        m = jnp.min(scores, axis=1, keepdims=True)
        iota = jax.lax.broadcasted_iota(jnp.int32, scores.shape, 1)
        code = jnp.min(jnp.where(scores == m, iota, V),
                       axis=1, keepdims=True)
        oh = (iota == code).astype(jnp.float32)
        q = jax.lax.dot_general(
            oh, cb, (((1,), (0,)), ((), ())),
            precision=jax.lax.Precision.HIGHEST,
            preferred_element_type=jnp.float32)
        quant = quant + q
        r = r - q
        codes_ref[:, k:k + 1] = code
    quant_ref[...] = quant
    resid_ref[...] = r


@functools.partial(jax.jit, static_argnames=("interpret",))
def kernel(x, cb0, cb1, cb2, cb3, interpret=False):
    cb_spec = pl.BlockSpec((V, D), lambda i: (0, 0))
    codes, quantized, residual = pl.pallas_call(
        _rvq_kernel,
        grid=(N // BN,),
        in_specs=[pl.BlockSpec((BN, D), lambda i: (i, 0)),
                  cb_spec, cb_spec, cb_spec, cb_spec],
        out_specs=(
            pl.BlockSpec((BN, NUM_STAGES), lambda i: (i, 0)),
            pl.BlockSpec((BN, D), lambda i: (i, 0)),
            pl.BlockSpec((BN, D), lambda i: (i, 0)),
        ),
        out_shape=(
            jax.ShapeDtypeStruct((N, NUM_STAGES), jnp.int32),
            jax.ShapeDtypeStruct((N, D), jnp.float32),
            jax.ShapeDtypeStruct((N, D), jnp.float32),
        ),
        interpret=interpret,
    )(x, cb0, cb1, cb2, cb3)
    return codes, quantized, residual


# speedup vs baseline: 26.1960x; 1.3982x over previous
import functools

import jax
import jax.numpy as jnp
from jax.experimental import pallas as pl

N = 1024
V = 1024
D = 256
NUM_STAGES = 4
BN = 256


def _rvq_kernel(x_ref, cb0_ref, cb1_ref, cb2_ref, cb3_ref,
                codes_ref, quant_ref, resid_ref):
    r = x_ref[...]
    quant = jnp.zeros_like(r)
    cb_refs = (cb0_ref, cb1_ref, cb2_ref, cb3_ref)
    for k in range(NUM_STAGES):
        cb = cb_refs[k][...]
        cbn = jnp.sum(cb * cb, axis=1, keepdims=True)
        dots = jax.lax.dot_general(
            r, cb, (((1,), (1,)), ((), ())),
            precision=jax.lax.Precision.HIGHEST,
            preferred_element_type=jnp.float32)
        scores = jnp.transpose(cbn) - 2.0 * dots
        m = jnp.min(scores, axis=1, keepdims=True)
        iota = jax.lax.broadcasted_iota(jnp.int32, scores.shape, 1)
        code = jnp.min(jnp.where(scores == m, iota, V),
                       axis=1, keepdims=True)
        oh = (iota == code).astype(jnp.bfloat16)
        cb1 = cb.astype(jnp.bfloat16)
        rest = cb - cb1.astype(jnp.float32)
        cb2 = rest.astype(jnp.bfloat16)
        cb3 = (rest - cb2.astype(jnp.float32)).astype(jnp.bfloat16)
        q = jnp.zeros_like(r)
        for cbp in (cb1, cb2, cb3):
            q = q + jax.lax.dot_general(
                oh, cbp, (((1,), (0,)), ((), ())),
                preferred_element_type=jnp.float32)
        quant = quant + q
        r = r - q
        codes_ref[:, k:k + 1] = code
    quant_ref[...] = quant
    resid_ref[...] = r


@functools.partial(jax.jit, static_argnames=("interpret",))
def kernel(x, cb0, cb1, cb2, cb3, interpret=False):
    cb_spec = pl.BlockSpec((V, D), lambda i: (0, 0))
    codes, quantized, residual = pl.pallas_call(
        _rvq_kernel,
        grid=(N // BN,),
        in_specs=[pl.BlockSpec((BN, D), lambda i: (i, 0)),
                  cb_spec, cb_spec, cb_spec, cb_spec],
        out_specs=(
            pl.BlockSpec((BN, NUM_STAGES), lambda i: (i, 0)),
            pl.BlockSpec((BN, D), lambda i: (i, 0)),
            pl.BlockSpec((BN, D), lambda i: (i, 0)),
        ),
        out_shape=(
            jax.ShapeDtypeStruct((N, NUM_STAGES), jnp.int32),
            jax.ShapeDtypeStruct((N, D), jnp.float32),
            jax.ShapeDtypeStruct((N, D), jnp.float32),
        ),
        interpret=interpret,
    )(x, cb0, cb1, cb2, cb3)
    return codes, quantized, residual
